# Initial kernel scaffold; baseline (speedup 1.0000x reference)
import jax
import jax.numpy as jnp
from jax.experimental import pallas as pl


def kernel(xyz):
    raise NotImplementedError("write your pallas kernel here")


# speedup vs baseline: 5.7933x; 5.7933x over previous
import functools

import jax
import jax.numpy as jnp
from jax import lax
from jax.experimental import pallas as pl
from jax.experimental.pallas import tpu as pltpu
import jax.experimental.pallas.tpu_sc as plsc

B = 16
N = 8192
G = 256
K = 32
DPAD = 16

_IDX_BIG = 2**30


def _fps_body(x_ref, y_ref, z_ref, idx_ref):
    x = x_ref[...]
    y = y_ref[...]
    z = z_ref[...]
    col = lax.broadcasted_iota(jnp.int32, (B, N), 1)
    gcol = lax.broadcasted_iota(jnp.int32, (B, G), 1)

    def step(g, carry):
        far, dists, iacc = carry
        maskf = (col == far).astype(jnp.float32)
        cx = jnp.sum(x * maskf, axis=1, keepdims=True)
        cy = jnp.sum(y * maskf, axis=1, keepdims=True)
        cz = jnp.sum(z * maskf, axis=1, keepdims=True)
        dx = x - cx
        dy = y - cy
        dz = z - cz
        d = dx * dx + dy * dy + dz * dz
        dists = jnp.minimum(dists, d)
        m = jnp.max(dists, axis=1, keepdims=True)
        cand = jnp.where(dists == m, col, _IDX_BIG)
        far = jnp.min(cand, axis=1, keepdims=True)
        iacc = jnp.where(gcol == g + 1, far.astype(jnp.float32), iacc)
        return far, dists, iacc

    far0 = jnp.zeros((B, 1), jnp.int32)
    dists0 = jnp.full((B, N), 1e10, jnp.float32)
    iacc0 = jnp.zeros((B, G), jnp.float32)
    _, _, iacc = lax.fori_loop(0, G - 1, step, (far0, dists0, iacc0))
    idx_ref[...] = iacc.astype(jnp.int32)


def _run_fps(x, y, z):
    return pl.pallas_call(
        _fps_body,
        out_shape=jax.ShapeDtypeStruct((B, G), jnp.int32),
    )(x, y, z)


def _knn_body(d_ref, idx_ref):
    b = pl.program_id(0)
    d = d_ref[0]
    col = lax.broadcasted_iota(jnp.int32, (G, N), 1)
    kcol = lax.broadcasted_iota(jnp.int32, (G, K), 1)

    def step(s, carry):
        d, acc = carry
        m = jnp.min(d, axis=1, keepdims=True)
        cand = jnp.where(d == m, col, _IDX_BIG)
        j = jnp.min(cand, axis=1, keepdims=True)
        acc = jnp.where(kcol == s, j, acc)
        d = jnp.where(col == j, jnp.inf, d)
        return d, acc

    acc0 = jnp.zeros((G, K), jnp.int32)
    _, acc = lax.fori_loop(0, K, step, (d, acc0))
    idx_ref[...] = (acc + b * N)[None]


def _run_knn(d):
    return pl.pallas_call(
        _knn_body,
        grid=(B,),
        in_specs=[
            pl.BlockSpec((1, G, N), lambda b: (b, 0, 0)),
        ],
        out_specs=pl.BlockSpec((1, G, K), lambda b: (b, 0, 0)),
        out_shape=jax.ShapeDtypeStruct((B, G, K), jnp.int32),
        compiler_params=pltpu.CompilerParams(
            vmem_limit_bytes=60 * 1024 * 1024),
    )(d)


_SC_NUM_CORES = 2
_SC_NUM_SUBCORES = 16
_NW = _SC_NUM_CORES * _SC_NUM_SUBCORES
_ROWS = B * G * K
_RPW = _ROWS // _NW
_GPW = _RPW // K
_ICH = 128
_NCH = _RPW // _ICH


def _gather_body(table_hbm, ctable_hbm, idx_hbm, out_hbm,
                 idx_v, rows_v, c_v, sem):
    wid = lax.axis_index("s") * _SC_NUM_CORES + lax.axis_index("c")
    base = wid * _RPW
    pltpu.sync_copy(idx_hbm.at[pl.ds(wid * _NCH, _NCH)], idx_v)
    handles = []
    for j in range(_NCH):
        handles.append(
            pltpu.async_copy(
                table_hbm.at[idx_v.at[j]],
                rows_v.at[pl.ds(j * _ICH, _ICH)],
                sem,
            )
        )
    for h in handles:
        h.wait()
    pltpu.sync_copy(ctable_hbm.at[pl.ds(wid * _GPW, _GPW)], c_v)

    def group_body(g, _):
        c = c_v[g]
        for s in range(K):
            r = g * K + s
            rows_v[r] = rows_v[r] - c
        return 0

    lax.fori_loop(0, _GPW, group_body, 0)
    pltpu.sync_copy(rows_v, out_hbm.at[pl.ds(base, _RPW)])


@functools.partial(jax.jit, static_argnums=())
def _run_gather(table, ctable, idx2d):
    mesh = plsc.VectorSubcoreMesh(core_axis_name="c", subcore_axis_name="s")
    return pl.kernel(
        _gather_body,
        out_type=jax.ShapeDtypeStruct((_ROWS, DPAD), jnp.float32),
        mesh=mesh,
        scratch_types=[
            pltpu.VMEM((_NCH, _ICH), jnp.int32),
            pltpu.VMEM((_RPW, DPAD), jnp.float32),
            pltpu.VMEM((_GPW, DPAD), jnp.float32),
            pltpu.SemaphoreType.DMA,
        ],
        compiler_params=pltpu.CompilerParams(use_tc_tiling_on_sc=False),
    )(table, ctable, idx2d)


def kernel(xyz):
    x = xyz[:, :, 0]
    y = xyz[:, :, 1]
    z = xyz[:, :, 2]
    fps_idx = _run_fps(x, y, z)
    xyz3 = xyz[:, :, :3]
    center = jnp.take_along_axis(xyz3, fps_idx[:, :, None], axis=1)
    d = (jnp.sum(center ** 2, axis=-1, keepdims=True)
         + jnp.sum(xyz3 ** 2, axis=-1)[:, None, :]
         - 2.0 * jnp.einsum('bgd,bnd->bgn', center, xyz3))
    idx = _run_knn(d)
    table = jnp.pad(xyz.reshape(B * N, 6), ((0, 0), (0, DPAD - 6)))
    ctable = jnp.pad(center.reshape(B * G, 3), ((0, 0), (0, DPAD - 3)))
    out = _run_gather(table, ctable, idx.reshape(_ROWS // _ICH, _ICH))
    out = out.reshape(B, G, K, DPAD)
    return out[..., :3], out[..., 3:6], center


# speedup vs baseline: 8.3385x; 1.4393x over previous
import functools

import jax
import jax.numpy as jnp
from jax import lax
from jax.experimental import pallas as pl
from jax.experimental.pallas import tpu as pltpu
import jax.experimental.pallas.tpu_sc as plsc

B = 16
N = 8192
G = 256
K = 32
DPAD = 16

_IDX_BIG = 2**30


def _fps_body(x_ref, y_ref, z_ref, idx_ref):
    x = x_ref[...]
    y = y_ref[...]
    z = z_ref[...]
    col = lax.broadcasted_iota(jnp.int32, (B, N), 1)
    gcol = lax.broadcasted_iota(jnp.int32, (B, G), 1)

    def step(g, carry):
        far, dists, iacc = carry
        maskf = (col == far).astype(jnp.float32)
        cx = jnp.sum(x * maskf, axis=1, keepdims=True)
        cy = jnp.sum(y * maskf, axis=1, keepdims=True)
        cz = jnp.sum(z * maskf, axis=1, keepdims=True)
        dx = x - cx
        dy = y - cy
        dz = z - cz
        d = dx * dx + dy * dy + dz * dz
        dists = jnp.minimum(dists, d)
        m = jnp.max(dists, axis=1, keepdims=True)
        cand = jnp.where(dists == m, col, _IDX_BIG)
        far = jnp.min(cand, axis=1, keepdims=True)
        iacc = jnp.where(gcol == g + 1, far.astype(jnp.float32), iacc)
        return far, dists, iacc

    far0 = jnp.zeros((B, 1), jnp.int32)
    dists0 = jnp.full((B, N), 1e10, jnp.float32)
    iacc0 = jnp.zeros((B, G), jnp.float32)
    _, _, iacc = lax.fori_loop(0, G - 1, step, (far0, dists0, iacc0))
    idx_ref[...] = iacc.astype(jnp.int32)


def _run_fps(x, y, z):
    return pl.pallas_call(
        _fps_body,
        out_shape=jax.ShapeDtypeStruct((B, G), jnp.int32),
    )(x, y, z)


def _knn_body(d_ref, idx_ref, ds_ref):
    b = pl.program_id(0)
    ds_ref[...] = d_ref[0]
    col = lax.broadcasted_iota(jnp.int32, (G, N), 1)
    kcol = lax.broadcasted_iota(jnp.int32, (G, K), 1)

    def step(s, acc):
        d = ds_ref[...]
        m = jnp.min(d, axis=1, keepdims=True)
        cand = jnp.where(d == m, col, _IDX_BIG)
        j = jnp.min(cand, axis=1, keepdims=True)
        acc = jnp.where(kcol == s, j, acc)
        ds_ref[...] = jnp.where(col == j, jnp.inf, d)
        return acc

    acc0 = jnp.zeros((G, K), jnp.int32)
    acc = lax.fori_loop(0, K, step, acc0)
    idx_ref[...] = (acc + b * N)[None]


def _run_knn(d):
    return pl.pallas_call(
        _knn_body,
        grid=(B,),
        in_specs=[
            pl.BlockSpec((1, G, N), lambda b: (b, 0, 0)),
        ],
        out_specs=pl.BlockSpec((1, G, K), lambda b: (b, 0, 0)),
        out_shape=jax.ShapeDtypeStruct((B, G, K), jnp.int32),
        scratch_shapes=[pltpu.VMEM((G, N), jnp.float32)],
        compiler_params=pltpu.CompilerParams(
            vmem_limit_bytes=60 * 1024 * 1024),
    )(d)


_SC_NUM_CORES = 2
_SC_NUM_SUBCORES = 16
_NW = _SC_NUM_CORES * _SC_NUM_SUBCORES
_ROWS = B * G * K
_RPW = _ROWS // _NW
_GPW = _RPW // K
_ICH = 128
_NCH = _RPW // _ICH


def _gather_body(table_hbm, ctable_hbm, idx_hbm, out_hbm,
                 idx_v, rows_v, c_v, sem):
    wid = lax.axis_index("s") * _SC_NUM_CORES + lax.axis_index("c")
    base = wid * _RPW
    pltpu.sync_copy(idx_hbm.at[pl.ds(wid * _NCH, _NCH)], idx_v)
    handles = []
    for j in range(_NCH):
        handles.append(
            pltpu.async_copy(
                table_hbm.at[idx_v.at[j]],
                rows_v.at[pl.ds(j * _ICH, _ICH)],
                sem,
            )
        )
    for h in handles:
        h.wait()
    pltpu.sync_copy(ctable_hbm.at[pl.ds(wid * _GPW, _GPW)], c_v)

    def group_body(g, _):
        c = c_v[g]
        for s in range(K):
            r = g * K + s
            rows_v[r] = rows_v[r] - c
        return 0

    lax.fori_loop(0, _GPW, group_body, 0)
    pltpu.sync_copy(rows_v, out_hbm.at[pl.ds(base, _RPW)])


@functools.partial(jax.jit, static_argnums=())
def _run_gather(table, ctable, idx2d):
    mesh = plsc.VectorSubcoreMesh(core_axis_name="c", subcore_axis_name="s")
    return pl.kernel(
        _gather_body,
        out_type=jax.ShapeDtypeStruct((_ROWS, DPAD), jnp.float32),
        mesh=mesh,
        scratch_types=[
            pltpu.VMEM((_NCH, _ICH), jnp.int32),
            pltpu.VMEM((_RPW, DPAD), jnp.float32),
            pltpu.VMEM((_GPW, DPAD), jnp.float32),
            pltpu.SemaphoreType.DMA,
        ],
        compiler_params=pltpu.CompilerParams(use_tc_tiling_on_sc=False),
    )(table, ctable, idx2d)


def kernel(xyz):
    x = xyz[:, :, 0]
    y = xyz[:, :, 1]
    z = xyz[:, :, 2]
    fps_idx = _run_fps(x, y, z)
    xyz3 = xyz[:, :, :3]
    center = jnp.take_along_axis(xyz3, fps_idx[:, :, None], axis=1)
    d = (jnp.sum(center ** 2, axis=-1, keepdims=True)
         + jnp.sum(xyz3 ** 2, axis=-1)[:, None, :]
         - 2.0 * jnp.einsum('bgd,bnd->bgn', center, xyz3))
    idx = _run_knn(d)
    table = jnp.pad(xyz.reshape(B * N, 6), ((0, 0), (0, DPAD - 6)))
    ctable = jnp.pad(center.reshape(B * G, 3), ((0, 0), (0, DPAD - 3)))
    out = _run_gather(table, ctable, idx.reshape(_ROWS // _ICH, _ICH))
    out = out.reshape(B, G, K, DPAD)
    return out[..., :3], out[..., 3:6], center
